# baseline (speedup 1.0000x reference)
import functools

import jax
import jax.numpy as jnp
from jax import lax
from jax.experimental import pallas as pl
from jax.experimental.pallas import tpu as pltpu
from jax.experimental.pallas import tpu_sc as plsc

VOCAB = 1000000
D = 64
B = 4096 * 200

NC = 2
NS = 16
NW = NC * NS
PW = B // NW
CHUNK = 800
NCHUNK = PW // CHUNK
LANES = 16


def _sc_body(x_hbm, table_hbm, out_hbm, idx_v, rows_v, gsem):
    wid = lax.axis_index("s") * NC + lax.axis_index("c")
    base = wid * PW

    pltpu.sync_copy(x_hbm.at[pl.ds(base, PW)], idx_v)

    def chunk_body(c, _):
        pltpu.async_copy(
            table_hbm.at[idx_v.at[pl.ds(c * CHUNK, CHUNK)]], rows_v, gsem
        ).wait()

        def row_body(r, _):
            for j in range(D // LANES):
                v = rows_v[r, pl.ds(j * LANES, LANES)]
                rows_v[r, pl.ds(j * LANES, LANES)] = jnp.sign(v)
            return 0

        lax.fori_loop(0, CHUNK, row_body, 0)
        pltpu.sync_copy(rows_v, out_hbm.at[pl.ds(base + c * CHUNK, CHUNK)])
        return 0

    lax.fori_loop(0, NCHUNK, chunk_body, 0)


@functools.partial(jax.jit, static_argnames=())
def kernel(x, table):
    x_flat = x.reshape(-1)
    mesh = plsc.VectorSubcoreMesh(core_axis_name="c", subcore_axis_name="s")
    out = pl.kernel(
        _sc_body,
        mesh=mesh,
        compiler_params=pltpu.CompilerParams(use_tc_tiling_on_sc=False),
        out_type=jax.ShapeDtypeStruct((B, D), jnp.float32),
        scratch_types=[
            pltpu.VMEM((PW,), jnp.int32),
            pltpu.VMEM((CHUNK, D), jnp.float32),
            pltpu.SemaphoreType.DMA,
        ],
    )(x_flat, table)
    return out.reshape(x.shape + (D,))


# speedup vs baseline: 1.1204x; 1.1204x over previous
import functools

import jax
import jax.numpy as jnp
from jax import lax
from jax.experimental import pallas as pl
from jax.experimental.pallas import tpu as pltpu
from jax.experimental.pallas import tpu_sc as plsc

VOCAB = 1000000
D = 64
B = 4096 * 200

NC = 2
NS = 16
NW = NC * NS
PW = B // NW
CHUNK = 400
NCHUNK = PW // CHUNK
NBUF = 4
GDEPTH = 2
LANES = 16
T_OUTER = NCHUNK // NBUF


def _sc_body(x_hbm, table_hbm, out_hbm, idx_v, rows_v, gsem, osem):
    wid = lax.axis_index("s") * NC + lax.axis_index("c")
    base = wid * PW

    def gather_issue(c, b):
        pltpu.async_copy(
            table_hbm.at[idx_v.at[pl.ds(c * CHUNK, CHUNK)]], rows_v.at[b],
            gsem.at[b])

    def gather_wait(b):
        pltpu.make_async_copy(
            table_hbm.at[idx_v.at[pl.ds(0, CHUNK)]], rows_v.at[b],
            gsem.at[b]).wait()

    def wo_issue(c, b):
        pltpu.async_copy(
            rows_v.at[b], out_hbm.at[pl.ds(base + c * CHUNK, CHUNK)],
            osem.at[b])

    def wo_wait(b):
        pltpu.make_async_copy(
            rows_v.at[b], out_hbm.at[pl.ds(base, CHUNK)], osem.at[b]).wait()

    def compute(b):
        def rbody(r, _):
            for j in range(D // LANES):
                v = rows_v[b, r, pl.ds(j * LANES, LANES)]
                rows_v[b, r, pl.ds(j * LANES, LANES)] = jnp.sign(v)
            return 0
        lax.fori_loop(0, CHUNK, rbody, 0)

    pltpu.sync_copy(x_hbm.at[pl.ds(base, PW)], idx_v)

    gather_issue(0, 0)
    gather_issue(1, 1)
    for u in range(NBUF):
        gather_wait(u)
        if u + GDEPTH >= NBUF:
            wo_wait((u + GDEPTH) % NBUF)
        gather_issue(u + GDEPTH, (u + GDEPTH) % NBUF)
        compute(u)
        wo_issue(u, u)

    def outer(t, _):
        c0 = t * NBUF
        for u in range(NBUF):
            gather_wait(u)
            b2 = (u + GDEPTH) % NBUF
            wo_wait(b2)
            gather_issue(c0 + u + GDEPTH, b2)
            compute(u)
            wo_issue(c0 + u, u)
        return 0

    lax.fori_loop(1, T_OUTER - 1, outer, 0)

    c0 = (T_OUTER - 1) * NBUF
    for u in range(NBUF):
        c = c0 + u
        gather_wait(u)
        if c + GDEPTH < NCHUNK:
            b2 = (u + GDEPTH) % NBUF
            wo_wait(b2)
            gather_issue(c + GDEPTH, b2)
        compute(u)
        wo_issue(c, u)

    for u in range(NBUF):
        wo_wait(u)


@functools.partial(jax.jit, static_argnames=())
def kernel(x, table):
    x_flat = x.reshape(-1)
    mesh = plsc.VectorSubcoreMesh(core_axis_name="c", subcore_axis_name="s")
    out = pl.kernel(
        _sc_body,
        mesh=mesh,
        compiler_params=pltpu.CompilerParams(use_tc_tiling_on_sc=False),
        out_type=jax.ShapeDtypeStruct((B, D), jnp.float32),
        scratch_types=[
            pltpu.VMEM((PW,), jnp.int32),
            pltpu.VMEM((NBUF, CHUNK, D), jnp.float32),
            pltpu.SemaphoreType.DMA((NBUF,)),
            pltpu.SemaphoreType.DMA((NBUF,)),
        ],
    )(x_flat, table)
    return out.reshape(x.shape + (D,))
